# baseline (speedup 1.0000x reference)
import jax
import jax.numpy as jnp
from jax.experimental import pallas as pl


def _moe_body(x_ref, g_ref, w_ref, b_ref, o_ref):
    x = x_ref[...].astype(jnp.bfloat16)
    g = jnp.maximum(g_ref[...], 0.0)
    n = w_ref.shape[0]
    acc = jax.lax.dot_general(
        g.astype(jnp.bfloat16), b_ref[...],
        (((1,), (0,)), ((), ())),
        preferred_element_type=jnp.float32,
    )
    for i in range(n):
        y = jax.lax.dot_general(
            x, w_ref[i],
            (((1,), (0,)), ((), ())),
            preferred_element_type=jnp.float32,
        )
        acc = acc + g[:, i:i + 1] * y
    o_ref[...] = acc


def kernel(input, gates, W, b):
    in_shape = input.shape
    d_in = in_shape[-1]
    n = gates.shape[-1]
    d_out = W.shape[-1]
    x = jnp.reshape(input, (-1, d_in))
    g = jnp.reshape(gates, (-1, n))
    t = x.shape[0]

    tb = 256
    if t % tb != 0:
        tb = t
    grid = (t // tb,)

    w_bf16 = W.astype(jnp.bfloat16)
    b_bf16 = b.astype(jnp.bfloat16)

    out = pl.pallas_call(
        _moe_body,
        grid=grid,
        in_specs=[
            pl.BlockSpec((tb, d_in), lambda i: (i, 0)),
            pl.BlockSpec((tb, n), lambda i: (i, 0)),
            pl.BlockSpec((n, d_in, d_out), lambda i: (0, 0, 0)),
            pl.BlockSpec((n, d_out), lambda i: (0, 0)),
        ],
        out_specs=pl.BlockSpec((tb, d_out), lambda i: (i, 0)),
        out_shape=jax.ShapeDtypeStruct((t, d_out), jnp.float32),
    )(x, g, w_bf16, b_bf16)

    return jnp.reshape(out, tuple(in_shape[:-1]) + (d_out,))


# speedup vs baseline: 1.0742x; 1.0742x over previous
import jax
import jax.numpy as jnp
from jax.experimental import pallas as pl
from jax.experimental.pallas import tpu as pltpu


def _moe_body(x_ref, g_ref, w_ref, b_ref, o_ref):
    x = x_ref[...].astype(jnp.bfloat16)
    g = jnp.maximum(g_ref[...], 0.0)
    n = w_ref.shape[0]
    acc = jax.lax.dot_general(
        g.astype(jnp.bfloat16), b_ref[...],
        (((1,), (0,)), ((), ())),
        preferred_element_type=jnp.float32,
    )
    for i in range(n):
        y = jax.lax.dot_general(
            x, w_ref[i],
            (((1,), (0,)), ((), ())),
            preferred_element_type=jnp.float32,
        )
        acc = acc + g[:, i:i + 1] * y
    o_ref[...] = acc


def kernel(input, gates, W, b):
    in_shape = input.shape
    d_in = in_shape[-1]
    n = gates.shape[-1]
    d_out = W.shape[-1]
    x = jnp.reshape(input, (-1, d_in))
    g = jnp.reshape(gates, (-1, n))
    t = x.shape[0]

    tb = 512
    if t % tb != 0:
        tb = t
    grid = (t // tb,)

    w_bf16 = W.astype(jnp.bfloat16)
    b_bf16 = b.astype(jnp.bfloat16)

    out = pl.pallas_call(
        _moe_body,
        grid=grid,
        in_specs=[
            pl.BlockSpec((tb, d_in), lambda i: (i, 0)),
            pl.BlockSpec((tb, n), lambda i: (i, 0)),
            pl.BlockSpec((n, d_in, d_out), lambda i: (0, 0, 0)),
            pl.BlockSpec((n, d_out), lambda i: (0, 0)),
        ],
        out_specs=pl.BlockSpec((tb, d_out), lambda i: (i, 0)),
        out_shape=jax.ShapeDtypeStruct((t, d_out), jnp.float32),
        compiler_params=pltpu.CompilerParams(
            dimension_semantics=("parallel",),
            allow_input_fusion=[False, False, True, True],
        ),
    )(x, g, w_bf16, b_bf16)

    return jnp.reshape(out, tuple(in_shape[:-1]) + (d_out,))


# speedup vs baseline: 1.0952x; 1.0196x over previous
import jax
import jax.numpy as jnp
from jax.experimental import pallas as pl
from jax.experimental.pallas import tpu as pltpu


def _moe_body(x_ref, g_ref, w_ref, b_ref, o_ref):
    x = x_ref[...].astype(jnp.bfloat16)
    g = jnp.maximum(g_ref[...], 0.0)
    n = w_ref.shape[0]
    acc = jax.lax.dot_general(
        g.astype(jnp.bfloat16), b_ref[...],
        (((1,), (0,)), ((), ())),
        preferred_element_type=jnp.float32,
    )
    for i in range(n):
        y = jax.lax.dot_general(
            x, w_ref[i],
            (((1,), (0,)), ((), ())),
            preferred_element_type=jnp.float32,
        )
        acc = acc + g[:, i:i + 1] * y
    o_ref[...] = acc


def kernel(input, gates, W, b):
    in_shape = input.shape
    d_in = in_shape[-1]
    n = gates.shape[-1]
    d_out = W.shape[-1]
    x = jnp.reshape(input, (-1, d_in))
    g = jnp.reshape(gates, (-1, n))
    t = x.shape[0]

    tb = 1024
    if t % tb != 0:
        tb = t
    grid = (t // tb,)

    w_bf16 = W.astype(jnp.bfloat16)
    b_bf16 = b.astype(jnp.bfloat16)

    out = pl.pallas_call(
        _moe_body,
        grid=grid,
        in_specs=[
            pl.BlockSpec((tb, d_in), lambda i: (i, 0)),
            pl.BlockSpec((tb, n), lambda i: (i, 0)),
            pl.BlockSpec((n, d_in, d_out), lambda i: (0, 0, 0)),
            pl.BlockSpec((n, d_out), lambda i: (0, 0)),
        ],
        out_specs=pl.BlockSpec((tb, d_out), lambda i: (i, 0)),
        out_shape=jax.ShapeDtypeStruct((t, d_out), jnp.float32),
        compiler_params=pltpu.CompilerParams(
            dimension_semantics=("parallel",),
            allow_input_fusion=[False, False, True, True],
        ),
    )(x, g, w_bf16, b_bf16)

    return jnp.reshape(out, tuple(in_shape[:-1]) + (d_out,))
